# baseline (speedup 1.0000x reference)
import functools
import math

import jax
import jax.numpy as jnp
from jax import lax
from jax.experimental import pallas as pl
from jax.experimental.pallas import tpu as pltpu
from jax.experimental.pallas import tpu_sc as plsc

N_NODES = 10000
N_EDGES = 320000
INPUT_SIZE = 128
HEAD_SIZE = 32
INV_DK = 1.0 / math.sqrt(HEAD_SIZE)

NC = 2
NS = 16
NW = NC * NS
LANES = 16

NPAD = 10240
EPAD = 327680
EW = EPAD // NW
CH = 128
NCHUNK = EW // CH
GROUPS = CH // LANES

D = HEAD_SIZE
ROWS_PER_TILE = NPAD // NS
ZROWS = 128


def _qkv_body(x_ref, wq_ref, wk_ref, wv_ref, ei_ref,
              q_ref, k_ref, v_ref, src_ref, dst_ref):
    x = x_ref[...]
    q_ref[...] = jnp.dot(x, wq_ref[...], preferred_element_type=jnp.float32)
    k_ref[...] = jnp.dot(x, wk_ref[...], preferred_element_type=jnp.float32)
    v_ref[...] = jnp.dot(x, wv_ref[...], preferred_element_type=jnp.float32)
    i = pl.program_id(0)
    eblk = src_ref.shape[0]
    gidx = (jax.lax.broadcasted_iota(jnp.int32, (1, eblk), 1)
            + i * eblk)
    padv = (gidx - N_EDGES) % (NPAD - N_NODES) + N_NODES
    real = gidx < N_EDGES
    src_ref[...] = jnp.where(real, ei_ref[0:1, :], padv).reshape(eblk)
    dst_ref[...] = jnp.where(real, ei_ref[1:2, :], padv).reshape(eblk)


def _qkv(x, wq, wk, wv, ei):
    blk = 2048
    w_spec = pl.BlockSpec((INPUT_SIZE, D), lambda i: (0, 0))
    o_spec = pl.BlockSpec((blk, D), lambda i: (i, 0))
    o_shape = jax.ShapeDtypeStruct((NPAD, D), jnp.float32)
    eblk = EPAD // (NPAD // blk)
    e_spec = pl.BlockSpec((2, eblk), lambda i: (0, i))
    e_ospec = pl.BlockSpec((eblk,), lambda i: (i,))
    e_oshape = jax.ShapeDtypeStruct((EPAD,), jnp.int32)
    return pl.pallas_call(
        _qkv_body,
        grid=(NPAD // blk,),
        in_specs=[pl.BlockSpec((blk, INPUT_SIZE), lambda i: (i, 0)),
                  w_spec, w_spec, w_spec, e_spec],
        out_specs=[o_spec, o_spec, o_spec, e_ospec, e_ospec],
        out_shape=[o_shape, o_shape, o_shape, e_oshape, e_oshape],
    )(x, wq, wk, wv, ei)


NBUF = 4

_SPLAT_DNUMS = lax.GatherDimensionNumbers(
    offset_dims=(), collapsed_slice_dims=(0,), start_index_map=(0,))


def _splat(x, idx):
    return lax.gather(x, idx[:, None], _SPLAT_DNUMS, (1,),
                      mode=lax.GatherScatterMode.PROMISE_IN_BOUNDS)


def _sc_edge_body(q_hbm, k_hbm, v_hbm, src_hbm, dst_hbm,
                  outp_hbm, den_hbm,
                  srcv_all, dstv_all, sidx, qr, kr, vr, wv, den_local, zb,
                  sems_g, sems_s, out_shared):
    c = lax.axis_index("c")
    s = lax.axis_index("s")
    wid = c * NS + s
    iota = lax.iota(jnp.int32, LANES)
    z16 = jnp.zeros((LANES,), jnp.float32)

    @pl.loop(0, ZROWS)
    def _(i):
        zb[i, pl.ds(0, LANES)] = z16
        zb[i, pl.ds(LANES, LANES)] = z16

    @pl.loop(0, NPAD // LANES)
    def _(i):
        den_local[pl.ds(i * LANES, LANES)] = z16

    @pl.loop(0, ROWS_PER_TILE // ZROWS)
    def _(i):
        pltpu.sync_copy(
            zb, out_shared.at[pl.ds(s * ROWS_PER_TILE + i * ZROWS, ZROWS)])

    plsc.subcore_barrier()

    pltpu.sync_copy(src_hbm.at[pl.ds(wid * EW, EW)], srcv_all)
    pltpu.sync_copy(dst_hbm.at[pl.ds(wid * EW, EW)], dstv_all)

    def fire_gathers(t, b):
        pltpu.async_copy(q_hbm.at[srcv_all.at[pl.ds(t * CH, CH)]],
                         qr.at[b], sems_g.at[b])
        pltpu.async_copy(k_hbm.at[dstv_all.at[pl.ds(t * CH, CH)]],
                         kr.at[b], sems_g.at[b])
        pltpu.async_copy(v_hbm.at[dstv_all.at[pl.ds(t * CH, CH)]],
                         vr.at[b], sems_g.at[b])

    def drain_gathers(b):
        pltpu.make_async_copy(q_hbm.at[srcv_all.at[pl.ds(0, CH)]], qr.at[b],
                              sems_g.at[b]).wait()
        pltpu.make_async_copy(k_hbm.at[dstv_all.at[pl.ds(0, CH)]], kr.at[b],
                              sems_g.at[b]).wait()
        pltpu.make_async_copy(v_hbm.at[dstv_all.at[pl.ds(0, CH)]], vr.at[b],
                              sems_g.at[b]).wait()

    def drain_scatter(b):
        pltpu.make_async_copy(wv.at[b], out_shared.at[sidx.at[b, 0]],
                              sems_s.at[b]).wait()

    lane15 = jnp.full((LANES,), 15, jnp.int32)

    def do_group(qref, kref, vref, wvref, g, srcg):
        pvec = z16
        for e in range(LANES):
            row = g * LANES + e
            q0 = qref[row, pl.ds(0, LANES)]
            q1 = qref[row, pl.ds(LANES, LANES)]
            k0 = kref[row, pl.ds(0, LANES)]
            k1 = kref[row, pl.ds(LANES, LANES)]
            w = q0 * k0 + q1 * k1
            wsum = jnp.cumsum(w)
            pe = jnp.exp(_splat(wsum, lane15) * INV_DK)
            pvec = jnp.where(iota == e, pe, pvec)
            v0 = vref[row, pl.ds(0, LANES)]
            v1 = vref[row, pl.ds(LANES, LANES)]
            wvref[row, pl.ds(0, LANES)] = v0 * pe
            wvref[row, pl.ds(LANES, LANES)] = v1 * pe
        plsc.addupdate_scatter(den_local, [srcg], pvec)

    for b in range(NBUF):
        fire_gathers(b, b)

    @pl.loop(0, NCHUNK // NBUF)
    def _(u):
        for b in range(NBUF):
            t = u * NBUF + b
            drain_gathers(b)

            @pl.when(t >= NBUF)
            def _():
                drain_scatter(b)

            @plsc.parallel_loop(0, GROUPS)
            def _(g):
                srcg = srcv_all[pl.ds(t * CH + g * LANES, LANES)]
                sidx[b, 0, pl.ds(g * LANES, LANES)] = srcg
                do_group(qr.at[b], kr.at[b], vr.at[b], wv.at[b], g, srcg)

            pltpu.async_copy(wv.at[b], out_shared.at[sidx.at[b, 0]],
                             sems_s.at[b], add=True)

            @pl.when(t + NBUF < NCHUNK)
            def _():
                fire_gathers(t + NBUF, b)

    for b in range(NBUF):
        drain_scatter(b)

    plsc.subcore_barrier()
    pltpu.sync_copy(out_shared.at[pl.ds(s * ROWS_PER_TILE, ROWS_PER_TILE)],
                    outp_hbm.at[c, pl.ds(s * ROWS_PER_TILE, ROWS_PER_TILE)])
    pltpu.sync_copy(den_local, den_hbm.at[wid])


def _sc_edge(q, k, v, src_p, dst_p):
    mesh = plsc.VectorSubcoreMesh(core_axis_name="c", subcore_axis_name="s")
    cp = pltpu.CompilerParams(needs_layout_passes=False,
                              use_tc_tiling_on_sc=False)
    f = pl.kernel(
        _sc_edge_body,
        out_type=[jax.ShapeDtypeStruct((NC, NPAD, D), jnp.float32),
                  jax.ShapeDtypeStruct((NW, NPAD), jnp.float32)],
        mesh=mesh,
        scratch_types=[
            pltpu.VMEM((EW,), jnp.int32),
            pltpu.VMEM((EW,), jnp.int32),
            pltpu.VMEM((NBUF, 1, CH), jnp.int32),
            pltpu.VMEM((NBUF, CH, D), jnp.float32),
            pltpu.VMEM((NBUF, CH, D), jnp.float32),
            pltpu.VMEM((NBUF, CH, D), jnp.float32),
            pltpu.VMEM((NBUF, CH, D), jnp.float32),
            pltpu.VMEM((NPAD,), jnp.float32),
            pltpu.VMEM((ZROWS, D), jnp.float32),
            pltpu.SemaphoreType.DMA((NBUF,)),
            pltpu.SemaphoreType.DMA((NBUF,)),
            pltpu.VMEM_SHARED((NPAD, D), jnp.float32),
        ],
        compiler_params=cp,
    )
    return f(q, k, v, src_p, dst_p)


NROWS_W = NPAD // NW
NROWS_LAST = N_NODES - 31 * NROWS_W


def _sc_norm_body(outp_hbm, den_hbm, o_hbm, o0, o1, dall, ov):
    c = lax.axis_index("c")
    s = lax.axis_index("s")
    wid = c * NS + s
    base = wid * NROWS_W
    pltpu.sync_copy(outp_hbm.at[0, pl.ds(base, NROWS_W)], o0)
    pltpu.sync_copy(outp_hbm.at[1, pl.ds(base, NROWS_W)], o1)
    pltpu.sync_copy(den_hbm.at[:, pl.ds(base, NROWS_W)], dall)

    @plsc.parallel_loop(0, NROWS_W // LANES)
    def _(g):
        dsum = dall[0, pl.ds(g * LANES, LANES)]
        for j in range(1, NW):
            dsum = dsum + dall[j, pl.ds(g * LANES, LANES)]
        rec = 1.0 / dsum
        for e in range(LANES):
            row = g * LANES + e
            re = _splat(rec, jnp.full((LANES,), e, jnp.int32))
            ov[row, pl.ds(0, LANES)] = (
                o0[row, pl.ds(0, LANES)] + o1[row, pl.ds(0, LANES)]) * re
            ov[row, pl.ds(LANES, LANES)] = (
                o0[row, pl.ds(LANES, LANES)] + o1[row, pl.ds(LANES, LANES)]) * re

    @pl.when(wid < NW - 1)
    def _():
        pltpu.sync_copy(ov, o_hbm.at[pl.ds(base, NROWS_W)])

    @pl.when(wid == NW - 1)
    def _():
        pltpu.sync_copy(ov.at[pl.ds(0, NROWS_LAST)],
                        o_hbm.at[pl.ds(base, NROWS_LAST)])


def _normalize(outp, den):
    mesh = plsc.VectorSubcoreMesh(core_axis_name="c", subcore_axis_name="s")
    cp = pltpu.CompilerParams(needs_layout_passes=False,
                              use_tc_tiling_on_sc=False)
    f = pl.kernel(
        _sc_norm_body,
        out_type=jax.ShapeDtypeStruct((N_NODES, D), jnp.float32),
        mesh=mesh,
        scratch_types=[
            pltpu.VMEM((NROWS_W, D), jnp.float32),
            pltpu.VMEM((NROWS_W, D), jnp.float32),
            pltpu.VMEM((NW, NROWS_W), jnp.float32),
            pltpu.VMEM((NROWS_W, D), jnp.float32),
        ],
        compiler_params=cp,
    )
    return f(outp, den)


def kernel(X, edge_index, Wq, Wk, Wv):
    ei = edge_index.astype(jnp.int32)
    q, k, v, src_p, dst_p = _qkv(X, Wq, Wk, Wv, ei)
    outp, den = _sc_edge(q, k, v, src_p, dst_p)
    return _normalize(outp, den)


# speedup vs baseline: 1.0157x; 1.0157x over previous
import functools
import math

import jax
import jax.numpy as jnp
from jax import lax
from jax.experimental import pallas as pl
from jax.experimental.pallas import tpu as pltpu
from jax.experimental.pallas import tpu_sc as plsc

N_NODES = 10000
N_EDGES = 320000
INPUT_SIZE = 128
HEAD_SIZE = 32
INV_DK = 1.0 / math.sqrt(HEAD_SIZE)

NC = 2
NS = 16
NW = NC * NS
LANES = 16

NPAD = 10240
EW = N_EDGES // NW
CH = 128
NCHUNK = EW // CH
EMINI = EW - NCHUNK * CH
GROUPS = CH // LANES

D = HEAD_SIZE
ROWS_PER_TILE = NPAD // NS
ZROWS = 128


def _qkv_body(x_ref, wq_ref, wk_ref, wv_ref, q_ref, k_ref, v_ref):
    x = x_ref[...]
    q_ref[...] = jnp.dot(x, wq_ref[...], preferred_element_type=jnp.float32)
    k_ref[...] = jnp.dot(x, wk_ref[...], preferred_element_type=jnp.float32)
    v_ref[...] = jnp.dot(x, wv_ref[...], preferred_element_type=jnp.float32)


def _qkv(x, wq, wk, wv):
    blk = 2048
    w_spec = pl.BlockSpec((INPUT_SIZE, D), lambda i: (0, 0))
    o_spec = pl.BlockSpec((blk, D), lambda i: (i, 0))
    o_shape = jax.ShapeDtypeStruct((NPAD, D), jnp.float32)
    return pl.pallas_call(
        _qkv_body,
        grid=(NPAD // blk,),
        in_specs=[pl.BlockSpec((blk, INPUT_SIZE), lambda i: (i, 0)),
                  w_spec, w_spec, w_spec],
        out_specs=[o_spec, o_spec, o_spec],
        out_shape=[o_shape, o_shape, o_shape],
    )(x, wq, wk, wv)


NBUF = 3

_SPLAT_DNUMS = lax.GatherDimensionNumbers(
    offset_dims=(), collapsed_slice_dims=(0,), start_index_map=(0,))


def _splat(x, idx):
    return lax.gather(x, idx[:, None], _SPLAT_DNUMS, (1,),
                      mode=lax.GatherScatterMode.PROMISE_IN_BOUNDS)


def _sc_edge_body(q_hbm, k_hbm, v_hbm, ei_hbm,
                  outp_hbm, den_hbm,
                  srcv_all, dstv_all, sidx, qr, kr, vr, wv, den_local, zb,
                  qm, km, vm, wvm, sidxm, sems_g, sems_s, out_shared):
    c = lax.axis_index("c")
    s = lax.axis_index("s")
    wid = c * NS + s
    iota = lax.iota(jnp.int32, LANES)
    z16 = jnp.zeros((LANES,), jnp.float32)

    @pl.loop(0, ZROWS)
    def _(i):
        zb[i, pl.ds(0, LANES)] = z16
        zb[i, pl.ds(LANES, LANES)] = z16

    @pl.loop(0, NPAD // LANES)
    def _(i):
        den_local[pl.ds(i * LANES, LANES)] = z16

    @pl.loop(0, ROWS_PER_TILE // ZROWS)
    def _(i):
        pltpu.sync_copy(
            zb, out_shared.at[pl.ds(s * ROWS_PER_TILE + i * ZROWS, ZROWS)])

    plsc.subcore_barrier()

    pltpu.sync_copy(ei_hbm.at[0, pl.ds(wid * EW, NCHUNK * CH)], srcv_all)
    pltpu.sync_copy(ei_hbm.at[1, pl.ds(wid * EW, NCHUNK * CH)], dstv_all)

    def fire_gathers(t, b):
        pltpu.async_copy(q_hbm.at[srcv_all.at[pl.ds(t * CH, CH)]],
                         qr.at[b], sems_g.at[b])
        pltpu.async_copy(k_hbm.at[dstv_all.at[pl.ds(t * CH, CH)]],
                         kr.at[b], sems_g.at[b])
        pltpu.async_copy(v_hbm.at[dstv_all.at[pl.ds(t * CH, CH)]],
                         vr.at[b], sems_g.at[b])

    def drain_gathers(b):
        pltpu.make_async_copy(q_hbm.at[srcv_all.at[pl.ds(0, CH)]], qr.at[b],
                              sems_g.at[b]).wait()
        pltpu.make_async_copy(k_hbm.at[dstv_all.at[pl.ds(0, CH)]], kr.at[b],
                              sems_g.at[b]).wait()
        pltpu.make_async_copy(v_hbm.at[dstv_all.at[pl.ds(0, CH)]], vr.at[b],
                              sems_g.at[b]).wait()

    def drain_scatter(b):
        pltpu.make_async_copy(wv.at[b], out_shared.at[sidx.at[b, 0]],
                              sems_s.at[b]).wait()

    lane15 = jnp.full((LANES,), 15, jnp.int32)

    def do_group(qref, kref, vref, wvref, g, srcg):
        pvec = z16
        for e in range(LANES):
            row = g * LANES + e
            q0 = qref[row, pl.ds(0, LANES)]
            q1 = qref[row, pl.ds(LANES, LANES)]
            k0 = kref[row, pl.ds(0, LANES)]
            k1 = kref[row, pl.ds(LANES, LANES)]
            w = q0 * k0 + q1 * k1
            wsum = jnp.cumsum(w)
            pe = jnp.exp(_splat(wsum, lane15) * INV_DK)
            pvec = jnp.where(iota == e, pe, pvec)
            v0 = vref[row, pl.ds(0, LANES)]
            v1 = vref[row, pl.ds(LANES, LANES)]
            wvref[row, pl.ds(0, LANES)] = v0 * pe
            wvref[row, pl.ds(LANES, LANES)] = v1 * pe
        plsc.addupdate_scatter(den_local, [srcg], pvec)

    for b in range(NBUF):
        fire_gathers(b, b)

    @pl.loop(0, NCHUNK // NBUF)
    def _(u):
        for b in range(NBUF):
            t = u * NBUF + b
            drain_gathers(b)

            @pl.when(t >= NBUF)
            def _():
                drain_scatter(b)

            @plsc.parallel_loop(0, GROUPS)
            def _(g):
                srcg = srcv_all[pl.ds(t * CH + g * LANES, LANES)]
                sidx[b, 0, pl.ds(g * LANES, LANES)] = srcg
                do_group(qr.at[b], kr.at[b], vr.at[b], wv.at[b], g, srcg)

            pltpu.async_copy(wv.at[b], out_shared.at[sidx.at[b, 0]],
                             sems_s.at[b], add=True)

            @pl.when(t + NBUF < NCHUNK)
            def _():
                fire_gathers(t + NBUF, b)

    for b in range(NBUF):
        drain_scatter(b)

    mbase = wid * EW + NCHUNK * CH
    pltpu.sync_copy(ei_hbm.at[0, pl.ds(mbase, EMINI)], sidxm.at[0])
    pltpu.sync_copy(ei_hbm.at[1, pl.ds(mbase, EMINI)], sidxm.at[1])
    pltpu.async_copy(q_hbm.at[sidxm.at[0]], qm, sems_g.at[0])
    pltpu.async_copy(k_hbm.at[sidxm.at[1]], km, sems_g.at[0])
    pltpu.async_copy(v_hbm.at[sidxm.at[1]], vm, sems_g.at[0])
    pltpu.make_async_copy(q_hbm.at[sidxm.at[0]], qm, sems_g.at[0]).wait()
    pltpu.make_async_copy(k_hbm.at[sidxm.at[1]], km, sems_g.at[0]).wait()
    pltpu.make_async_copy(v_hbm.at[sidxm.at[1]], vm, sems_g.at[0]).wait()
    srcm = sidxm[0, pl.ds(0, EMINI)]
    do_group(qm, km, vm, wvm, 0, srcm)
    pltpu.sync_copy(wvm, out_shared.at[sidxm.at[0]], add=True)

    plsc.subcore_barrier()
    pltpu.sync_copy(out_shared.at[pl.ds(s * ROWS_PER_TILE, ROWS_PER_TILE)],
                    outp_hbm.at[c, pl.ds(s * ROWS_PER_TILE, ROWS_PER_TILE)])
    pltpu.sync_copy(den_local, den_hbm.at[wid])


def _sc_edge(q, k, v, ei):
    mesh = plsc.VectorSubcoreMesh(core_axis_name="c", subcore_axis_name="s")
    cp = pltpu.CompilerParams(needs_layout_passes=False,
                              use_tc_tiling_on_sc=False)
    f = pl.kernel(
        _sc_edge_body,
        out_type=[jax.ShapeDtypeStruct((NC, NPAD, D), jnp.float32),
                  jax.ShapeDtypeStruct((NW, NPAD), jnp.float32)],
        mesh=mesh,
        scratch_types=[
            pltpu.VMEM((NCHUNK * CH,), jnp.int32),
            pltpu.VMEM((NCHUNK * CH,), jnp.int32),
            pltpu.VMEM((NBUF, 1, CH), jnp.int32),
            pltpu.VMEM((NBUF, CH, D), jnp.float32),
            pltpu.VMEM((NBUF, CH, D), jnp.float32),
            pltpu.VMEM((NBUF, CH, D), jnp.float32),
            pltpu.VMEM((NBUF, CH, D), jnp.float32),
            pltpu.VMEM((NPAD,), jnp.float32),
            pltpu.VMEM((ZROWS, D), jnp.float32),
            pltpu.VMEM((EMINI, D), jnp.float32),
            pltpu.VMEM((EMINI, D), jnp.float32),
            pltpu.VMEM((EMINI, D), jnp.float32),
            pltpu.VMEM((EMINI, D), jnp.float32),
            pltpu.VMEM((2, EMINI), jnp.int32),
            pltpu.SemaphoreType.DMA((NBUF,)),
            pltpu.SemaphoreType.DMA((NBUF,)),
            pltpu.VMEM_SHARED((NPAD, D), jnp.float32),
        ],
        compiler_params=cp,
    )
    return f(q, k, v, ei)


NROWS_W = NPAD // NW
NROWS_LAST = N_NODES - 31 * NROWS_W


def _sc_norm_body(outp_hbm, den_hbm, o_hbm, o0, o1, dall, ov):
    c = lax.axis_index("c")
    s = lax.axis_index("s")
    wid = c * NS + s
    base = wid * NROWS_W
    pltpu.sync_copy(outp_hbm.at[0, pl.ds(base, NROWS_W)], o0)
    pltpu.sync_copy(outp_hbm.at[1, pl.ds(base, NROWS_W)], o1)
    pltpu.sync_copy(den_hbm.at[:, pl.ds(base, NROWS_W)], dall)

    @plsc.parallel_loop(0, NROWS_W // LANES)
    def _(g):
        dsum = dall[0, pl.ds(g * LANES, LANES)]
        for j in range(1, NW):
            dsum = dsum + dall[j, pl.ds(g * LANES, LANES)]
        rec = 1.0 / dsum
        for e in range(LANES):
            row = g * LANES + e
            re = _splat(rec, jnp.full((LANES,), e, jnp.int32))
            ov[row, pl.ds(0, LANES)] = (
                o0[row, pl.ds(0, LANES)] + o1[row, pl.ds(0, LANES)]) * re
            ov[row, pl.ds(LANES, LANES)] = (
                o0[row, pl.ds(LANES, LANES)] + o1[row, pl.ds(LANES, LANES)]) * re

    @pl.when(wid < NW - 1)
    def _():
        pltpu.sync_copy(ov, o_hbm.at[pl.ds(base, NROWS_W)])

    @pl.when(wid == NW - 1)
    def _():
        pltpu.sync_copy(ov.at[pl.ds(0, NROWS_LAST)],
                        o_hbm.at[pl.ds(base, NROWS_LAST)])


def _normalize(outp, den):
    mesh = plsc.VectorSubcoreMesh(core_axis_name="c", subcore_axis_name="s")
    cp = pltpu.CompilerParams(needs_layout_passes=False,
                              use_tc_tiling_on_sc=False)
    f = pl.kernel(
        _sc_norm_body,
        out_type=jax.ShapeDtypeStruct((N_NODES, D), jnp.float32),
        mesh=mesh,
        scratch_types=[
            pltpu.VMEM((NROWS_W, D), jnp.float32),
            pltpu.VMEM((NROWS_W, D), jnp.float32),
            pltpu.VMEM((NW, NROWS_W), jnp.float32),
            pltpu.VMEM((NROWS_W, D), jnp.float32),
        ],
        compiler_params=cp,
    )
    return f(outp, den)


def kernel(X, edge_index, Wq, Wk, Wv):
    ei = edge_index.astype(jnp.int32)
    q, k, v = _qkv(X, Wq, Wk, Wv)
    outp, den = _sc_edge(q, k, v, ei)
    return _normalize(outp, den)
